# Initial kernel scaffold; baseline (speedup 1.0000x reference)
import jax
import jax.numpy as jnp
from jax.experimental import pallas as pl


def kernel(tgt_index, knn_dists, knn_key_feature, network_probs, network_select_probs, W_func, b_func, W_fc1a, b_fc1a, W_fc1b, b_fc1b, W_fc2a, b_fc2a, W_fc2b, b_fc2b):
    raise NotImplementedError("write your pallas kernel here")


# speedup vs baseline: 1.1808x; 1.1808x over previous
import functools

import jax
import jax.numpy as jnp
from jax import lax
from jax.experimental import pallas as pl
from jax.experimental.pallas import tpu as pltpu
from jax.experimental.pallas import tpu_sc as plsc

_B, _S, _K, _V = 16, 8, 32, 100000
_R = _B * _S
_E = _R * _K
_TOT = _R * _V
_ZROWS, _ZCOLS = 25000, 512
_ZBLK = 1000
_NZ = _ZROWS // _ZBLK
_NC, _NS = 2, 16
_NW = _NC * _NS
_EPW = _E // _NW


def _tc_body(idx_ref, d_ref, kf_ref, nsp_ref, w1a_ref, b1a_ref, w1b_ref,
             b1b_ref, w2a_ref, b2a_ref, wt_ref, bt_ref,
             z_ref, gidx_ref, vals_ref):
    z_ref[...] = jnp.zeros_like(z_ref)

    @pl.when(pl.program_id(0) == 0)
    def _compute():
        idx = idx_ref[...]
        d = d_ref[...]
        eqf = jnp.where(idx[:, :, None] == idx[:, None, :], 1.0, 0.0)
        a0 = lax.broadcasted_iota(jnp.int32, (_K, _K), 0)
        a1 = lax.broadcasted_iota(jnp.int32, (_K, _K), 1)
        prior = jnp.sum(jnp.where((a1 < a0)[None], eqf, 0.0), axis=2)
        later = jnp.sum(jnp.where((a1 > a0)[None], eqf, 0.0), axis=2)
        new = jnp.where((prior == 0.0) & (idx != 0), 1.0, 0.0)
        ltri = jnp.where(a0 <= a1, 1.0, 0.0)
        counts = jnp.dot(new, ltri, preferred_element_type=jnp.float32)

        logkf = jnp.log(kf_ref[...])
        lognsp = jnp.log(nsp_ref[...])
        noise = jnp.zeros_like(d) + b1b_ref[0:1, 0:1]
        for c in range(4):
            h1c = jnp.tanh(logkf * w1a_ref[0:1, c:c + 1]
                           + lognsp * w1a_ref[1:2, c:c + 1]
                           + b1a_ref[0:1, c:c + 1])
            noise = noise + h1c * w1b_ref[0:1, c:c + 1]

        h2 = jnp.tanh(jnp.dot(d, w2a_ref[:_K, :],
                              preferred_element_type=jnp.float32)
                      + jnp.dot(counts, w2a_ref[_K:, :],
                                preferred_element_type=jnp.float32)
                      + b2a_ref[...])
        tlogit = jnp.sum(h2 * wt_ref[...], axis=1, keepdims=True) \
            + bt_ref[0:1, 0:1]
        tempe = jax.nn.sigmoid(tlogit)

        x = -d * tempe + noise
        m = jnp.max(x, axis=1, keepdims=True)
        e = jnp.exp(x - m)
        probs = e / jnp.sum(e, axis=1, keepdims=True)

        contrib = probs * jnp.where(later == 0.0, 1.0, 0.0)
        vals = jnp.sum(eqf * contrib[:, None, :], axis=2)

        rows = lax.broadcasted_iota(jnp.int32, (_R, _K), 0)
        gidx_ref[...] = rows * _V + idx
        vals_ref[...] = vals


_tc_call = pl.pallas_call(
    _tc_body,
    grid=(_NZ,),
    in_specs=[
        pl.BlockSpec((_R, _K), lambda i: (0, 0)),
        pl.BlockSpec((_R, _K), lambda i: (0, 0)),
        pl.BlockSpec((_R, _K), lambda i: (0, 0)),
        pl.BlockSpec((_R, _K), lambda i: (0, 0)),
        pl.BlockSpec((2, 4), lambda i: (0, 0)),
        pl.BlockSpec((1, 4), lambda i: (0, 0)),
        pl.BlockSpec((1, 4), lambda i: (0, 0)),
        pl.BlockSpec((1, 1), lambda i: (0, 0)),
        pl.BlockSpec((2 * _K, _K), lambda i: (0, 0)),
        pl.BlockSpec((1, _K), lambda i: (0, 0)),
        pl.BlockSpec((1, _K), lambda i: (0, 0)),
        pl.BlockSpec((1, 1), lambda i: (0, 0)),
    ],
    out_specs=[
        pl.BlockSpec((_ZBLK, _ZCOLS), lambda i: (i, 0)),
        pl.BlockSpec((_R, _K), lambda i: (0, 0)),
        pl.BlockSpec((_R, _K), lambda i: (0, 0)),
    ],
    out_shape=[
        jax.ShapeDtypeStruct((_ZROWS, _ZCOLS), jnp.float32),
        jax.ShapeDtypeStruct((_R, _K), jnp.int32),
        jax.ShapeDtypeStruct((_R, _K), jnp.float32),
    ],
)


def _sc_body(out_ref, gidx_ref, vals_ref, idx_v, val_v, sem):
    wid = lax.axis_index("s") * _NC + lax.axis_index("c")
    base = wid * _EPW
    pltpu.sync_copy(gidx_ref.at[pl.ds(base, _EPW)], idx_v)
    pltpu.sync_copy(vals_ref.at[pl.ds(base, _EPW)], val_v)
    pltpu.async_copy(val_v, out_ref.at[idx_v], sem).wait()


@functools.cache
def _sc_scatter():
    return pl.kernel(
        _sc_body,
        out_type=(),
        mesh=plsc.VectorSubcoreMesh(core_axis_name="c", subcore_axis_name="s"),
        scratch_types=[
            pltpu.VMEM((_EPW,), jnp.int32),
            pltpu.VMEM((_EPW,), jnp.float32),
            pltpu.SemaphoreType.DMA,
        ],
    )


def kernel(tgt_index, knn_dists, knn_key_feature, network_probs,
           network_select_probs, W_func, b_func, W_fc1a, b_fc1a, W_fc1b,
           b_fc1b, W_fc2a, b_fc2a, W_fc2b, b_fc2b):
    del network_probs, W_func, b_func
    idx2 = tgt_index.reshape(_R, _K).astype(jnp.int32)
    d2 = knn_dists.reshape(_R, _K)
    kf2 = knn_key_feature.reshape(_R, _K)
    nsp2 = network_select_probs.reshape(_R, _K)
    w1a = W_fc1a
    b1a = b_fc1a.reshape(1, 4)
    w1b = W_fc1b.reshape(1, 4)
    b1b = b_fc1b.reshape(1, 1)
    w2a = W_fc2a
    b2a = b_fc2a.reshape(1, _K)
    wt = W_fc2b[:, 1].reshape(1, _K)
    bt = b_fc2b[1].reshape(1, 1)

    zeros2d, gidx, vals = _tc_call(idx2, d2, kf2, nsp2, w1a, b1a, w1b, b1b,
                                   w2a, b2a, wt, bt)
    buf = jax.new_ref(zeros2d.reshape(_TOT))
    _sc_scatter()(buf, gidx.reshape(_E), vals.reshape(_E))
    return buf[...].reshape(_B, _S, _V)


# speedup vs baseline: 3.9274x; 3.3260x over previous
import functools

import jax
import jax.numpy as jnp
from jax import lax
from jax.experimental import pallas as pl
from jax.experimental.pallas import tpu as pltpu
from jax.experimental.pallas import tpu_sc as plsc

_B, _S, _K, _V = 16, 8, 32, 100000
_R = _B * _S
_E = _R * _K
_NC, _NS = 2, 16
_NW = _NC * _NS
_VT = (_V + 127) // 128
_BSTRIDE = _VT * 1024
_HT = _VT // 2
_CT = 23
_NCH = _HT // _CT
_CW = _CT * 128
_CWORDS = _CT * 1024


def _tc_body(idx_ref, d_ref, kf_ref, nsp_ref, w1a_ref, b1a_ref, w1b_ref,
             b1b_ref, w2a_ref, b2a_ref, w2b_ref, b2b_ref,
             pcode_ref, vals_ref):
    idx = idx_ref[...]
    d = d_ref[...]
    eqf = jnp.where(idx[:, :, None] == idx[:, None, :], 1.0, 0.0)
    a0 = lax.broadcasted_iota(jnp.int32, (_K, _K), 0)
    a1 = lax.broadcasted_iota(jnp.int32, (_K, _K), 1)
    prior = jnp.sum(jnp.where((a1 < a0)[None], eqf, 0.0), axis=2)
    later = jnp.sum(jnp.where((a1 > a0)[None], eqf, 0.0), axis=2)
    new = jnp.where((prior == 0.0) & (idx != 0), 1.0, 0.0)
    ltri = jnp.where(a0 <= a1, 1.0, 0.0)
    counts = jnp.dot(new, ltri, preferred_element_type=jnp.float32)

    def _bf(x):
        return x.astype(jnp.bfloat16).astype(jnp.float32)

    logkf = _bf(jnp.log(kf_ref[...]))
    lognsp = _bf(jnp.log(nsp_ref[...]))
    acc1 = jnp.zeros_like(d)
    for c in range(4):
        h1c = jnp.tanh(logkf * _bf(w1a_ref[0:1, c:c + 1])
                       + lognsp * _bf(w1a_ref[1:2, c:c + 1])
                       + b1a_ref[0:1, c:c + 1])
        acc1 = acc1 + _bf(h1c) * _bf(w1b_ref[c:c + 1, 0:1])
    noise = acc1 + b1b_ref[0:1, 0:1]

    bd = _bf(d)
    acc = jnp.zeros_like(d)
    for j in range(_K):
        acc = acc + bd[:, j:j + 1] * _bf(w2a_ref[j:j + 1, :])
    for j in range(_K):
        acc = acc + counts[:, j:j + 1] * _bf(w2a_ref[_K + j:_K + j + 1, :])
    h2 = jnp.tanh(acc + b2a_ref[...])
    bh2 = _bf(h2)
    bwt = _bf(w2b_ref[:, 1].reshape(1, _K))
    tlogit = jnp.sum(bh2 * bwt, axis=1, keepdims=True) + b2b_ref[0:1, 1:2]
    tempe = jax.nn.sigmoid(tlogit)

    x = -d * tempe + noise
    m = jnp.max(x, axis=1, keepdims=True)
    e = jnp.exp(x - m)
    probs = e / jnp.sum(e, axis=1, keepdims=True)

    contrib = probs * jnp.where(later == 0.0, 1.0, 0.0)
    vals = jnp.sum(eqf * contrib[:, None, :], axis=2)

    rows = lax.broadcasted_iota(jnp.int32, (_R, _K), 0)
    pcode_ref[...] = (rows & 7) * (1 << 20) + idx
    vals_ref[...] = vals


_tc_call = pl.pallas_call(
    _tc_body,
    out_shape=[
        jax.ShapeDtypeStruct((_R, _K), jnp.int32),
        jax.ShapeDtypeStruct((_R, _K), jnp.float32),
    ],
)


def _sc_body(pcode_ref, vals_ref, out_ref, zbuf, idx_v, val_v, zsem):
    wid = lax.axis_index("s") * _NC + lax.axis_index("c")
    b = wid >> 1
    h = wid & 1

    z16 = jnp.zeros((16,), jnp.float32)

    def _zero(i, carry):
        base = i * 16
        for u in range(8):
            zbuf[u, pl.ds(base, 16)] = z16
        return carry

    lax.fori_loop(0, _CW // 16, _zero, 0)

    pltpu.sync_copy(pcode_ref.at[pl.ds(8 * b, 8)], idx_v)
    pltpu.sync_copy(vals_ref.at[pl.ds(8 * b, 8)], val_v)
    hbase = h * (_HT * 128)
    ngrp = 2
    subs, vpos, vl_ = [], [], []
    for rr in range(8):
        for g in range(ngrp):
            p = idx_v[rr, pl.ds(g * 16, 16)]
            subs.append(p >> 20)
            vpos.append((p & ((1 << 20) - 1)) - hbase)
            vl_.append(val_v[rr, pl.ds(g * 16, 16)])

    def _splice(c0, restore):
        for g in range(8 * ngrp):
            col = vpos[g] - c0
            m = (col >= 0) & (col < _CW)
            coff = jnp.where(m, col, 0)
            v = z16 if restore else vl_[g]
            plsc.store_scatter(zbuf, [subs[g], coff], v, mask=m)

    for c in range(_NCH):
        c0 = c * _CW
        _splice(c0, False)
        pltpu.async_copy(
            zbuf, out_ref.at[b, :, pl.ds(hbase + c0, _CW)], zsem).wait()
        _splice(c0, True)


@functools.cache
def _sc_zero_scatter():
    return pl.kernel(
        _sc_body,
        out_type=jax.ShapeDtypeStruct((_B, _S, _V), jnp.float32),
        mesh=plsc.VectorSubcoreMesh(core_axis_name="c", subcore_axis_name="s"),
        compiler_params=pltpu.CompilerParams(needs_layout_passes=False),
        scratch_types=[
            pltpu.VMEM((8, _CW), jnp.float32),
            pltpu.VMEM((8, _K), jnp.int32),
            pltpu.VMEM((8, _K), jnp.float32),
            pltpu.SemaphoreType.DMA,
        ],
    )


def kernel(tgt_index, knn_dists, knn_key_feature, network_probs,
           network_select_probs, W_func, b_func, W_fc1a, b_fc1a, W_fc1b,
           b_fc1b, W_fc2a, b_fc2a, W_fc2b, b_fc2b):
    del network_probs, W_func, b_func
    idx2 = tgt_index.reshape(_R, _K).astype(jnp.int32)
    d2 = knn_dists.reshape(_R, _K)
    kf2 = knn_key_feature.reshape(_R, _K)
    nsp2 = network_select_probs.reshape(_R, _K)
    pcode, vals = _tc_call(
        idx2, d2, kf2, nsp2, W_fc1a, b_fc1a.reshape(1, 4),
        W_fc1b, b_fc1b.reshape(1, 1), W_fc2a,
        b_fc2a.reshape(1, _K), W_fc2b, b_fc2b.reshape(1, 2))
    return _sc_zero_scatter()(pcode, vals)
